# baseline (speedup 1.0000x reference)
import jax
import jax.numpy as jnp
from jax.experimental import pallas as pl


def kernel(inputs, pos_table):
    B, N, D = inputs.shape
    P = 256

    def body(x_ref, t_ref, o_ref):
        o_ref[...] = x_ref[...] + t_ref[...]

    return pl.pallas_call(
        body,
        grid=(N // P, B),
        in_specs=[
            pl.BlockSpec((1, P, D), lambda i, b: (b, i, 0)),
            pl.BlockSpec((P, D), lambda i, b: (i, 0)),
        ],
        out_specs=pl.BlockSpec((1, P, D), lambda i, b: (b, i, 0)),
        out_shape=jax.ShapeDtypeStruct((B, N, D), inputs.dtype),
    )(inputs, pos_table)
